# baseline (speedup 1.0000x reference)
import jax
import jax.numpy as jnp
from jax import lax
from jax.experimental import pallas as pl
from jax.experimental.pallas import tpu as pltpu
from jax.experimental.pallas import tpu_sc as plsc

_DIM = 1024
_HALF = _DIM // 2
_L0 = 64
_L1 = 64
_LANES = 16
_NVR = _HALF // _LANES
_NC = 2
_NS = 16
_NW = _NC * _NS
_C0W = _L0 // _NW
_ROWS_W = _C0W * _L1


def _tables_kernel(w0_ref, b0_ref, w1_ref, b1_ref, scal_ref,
                   emb0_ref, emb1_ref):
    c = jax.lax.broadcasted_iota(jnp.int32, (_L0, 1), 0).astype(jnp.float32)
    pos0 = c / scal_ref[0, 0] * scal_ref[0, 1]
    emb0_ref[...] = jnp.sin(pos0 * w0_ref[...] + b0_ref[...])
    pos1 = c / scal_ref[0, 2] * scal_ref[0, 3]
    emb1_ref[...] = jnp.sin(pos1 * w1_ref[...] + b1_ref[...])


def _sc_expand(emb0_hbm, emb1_hbm, out_hbm, emb0_v, bc0_v, bc1_v,
               shm_emb1, sem):
    cid = lax.axis_index("c")
    sid = lax.axis_index("s")
    wid = sid * _NC + cid
    @pl.when(sid == 0)
    def _():
        pltpu.sync_copy(emb1_hbm, shm_emb1)

    pltpu.sync_copy(emb0_hbm.at[pl.ds(wid * _C0W, _C0W)], emb0_v)
    for j, bc_v in ((0, bc0_v), (1, bc1_v)):
        row_vals = [emb0_v[j, pl.ds(v * _LANES, _LANES)] for v in range(_NVR)]

        def _fill(r2, carry, bc_v=bc_v, row_vals=row_vals):
            for rr in range(2):
                for v in range(_NVR):
                    bc_v[r2 * 2 + rr, pl.ds(v * _LANES, _LANES)] = row_vals[v]
            return carry

        lax.fori_loop(0, _L1 // 2, _fill, 0)
    plsc.subcore_barrier()

    base = wid * _ROWS_W
    for j in range(_C0W):
        blk = base + j * _L1
        pltpu.async_copy(
            shm_emb1, out_hbm.at[pl.ds(blk, _L1), pl.ds(_HALF, _HALF)], sem)
    for j, bc_v in ((0, bc0_v), (1, bc1_v)):
        blk = base + j * _L1
        pltpu.async_copy(
            bc_v, out_hbm.at[pl.ds(blk, _L1), pl.ds(0, _HALF)], sem)
    pltpu.make_async_copy(
        out_hbm.at[pl.ds(base, _ROWS_W)],
        out_hbm.at[pl.ds(base, _ROWS_W)],
        sem,
    ).wait()


def kernel(seq_len_or_axial_dims, W0, b0, W1, b1, div0, mult0, div1, mult1):
    w0 = W0.reshape(1, _HALF)
    b0r = b0.reshape(1, _HALF)
    w1 = W1.reshape(1, _HALF)
    b1r = b1.reshape(1, _HALF)
    scal = jnp.stack([div0, mult0, div1, mult1]).reshape(1, 4)

    emb0, emb1 = pl.pallas_call(
        _tables_kernel,
        out_shape=[
            jax.ShapeDtypeStruct((_L0, _HALF), jnp.float32),
            jax.ShapeDtypeStruct((_L1, _HALF), jnp.float32),
        ],
    )(w0, b0r, w1, b1r, scal)

    expand = pl.kernel(
        _sc_expand,
        out_type=jax.ShapeDtypeStruct((_L0 * _L1, _DIM), jnp.float32),
        mesh=plsc.VectorSubcoreMesh(core_axis_name="c", subcore_axis_name="s"),
        scratch_types=[
            pltpu.VMEM((_C0W, _HALF), jnp.float32),
            pltpu.VMEM((_L1, _HALF), jnp.float32),
            pltpu.VMEM((_L1, _HALF), jnp.float32),
            pltpu.VMEM_SHARED((_L1, _HALF), jnp.float32),
            pltpu.SemaphoreType.DMA,
        ],
    )
    return expand(emb0, emb1)


# speedup vs baseline: 1.0417x; 1.0417x over previous
import jax
import jax.numpy as jnp
from jax import lax
from jax.experimental import pallas as pl
from jax.experimental.pallas import tpu as pltpu
from jax.experimental.pallas import tpu_sc as plsc

_DIM = 1024
_HALF = _DIM // 2
_L0 = 64
_L1 = 64
_LANES = 16
_NVR = _HALF // _LANES
_NC = 2
_NS = 16
_NW = _NC * _NS
_C0W = _L0 // _NW
_ROWS_W = _C0W * _L1


def _tables_kernel(w0_ref, b0_ref, w1_ref, b1_ref, scal_ref,
                   emb0_ref, emb1_ref):
    c = jax.lax.broadcasted_iota(jnp.int32, (_L0, 1), 0).astype(jnp.float32)
    pos0 = c / scal_ref[0, 0] * scal_ref[0, 1]
    emb0_ref[...] = jnp.sin(pos0 * w0_ref[...] + b0_ref[...])
    pos1 = c / scal_ref[0, 2] * scal_ref[0, 3]
    emb1_ref[...] = jnp.sin(pos1 * w1_ref[...] + b1_ref[...])


def _sc_expand(emb0_hbm, emb1_hbm, out_hbm, emb0_v, bc0_v, bc1_v,
               shm_emb1, sem):
    cid = lax.axis_index("c")
    sid = lax.axis_index("s")
    wid = sid * _NC + cid
    @pl.when(sid < 8)
    def _():
        off = pl.multiple_of(sid * 8, 8)
        pltpu.sync_copy(emb1_hbm.at[pl.ds(off, 8)],
                        shm_emb1.at[pl.ds(off, 8)])

    pltpu.sync_copy(emb0_hbm.at[pl.ds(wid * _C0W, _C0W)], emb0_v)
    plsc.subcore_barrier()

    base = wid * _ROWS_W
    for j in range(_C0W):
        blk = base + j * _L1
        pltpu.async_copy(
            shm_emb1, out_hbm.at[pl.ds(blk, _L1), pl.ds(_HALF, _HALF)], sem)
    for j, bc_v in ((0, bc0_v), (1, bc1_v)):
        blk = base + j * _L1
        row_vals = [emb0_v[j, pl.ds(v * _LANES, _LANES)] for v in range(_NVR)]

        def _fill(r2, carry, bc_v=bc_v, row_vals=row_vals):
            for rr in range(2):
                for v in range(_NVR):
                    bc_v[r2 * 2 + rr, pl.ds(v * _LANES, _LANES)] = row_vals[v]
            return carry

        lax.fori_loop(0, _L1 // 2, _fill, 0)
        pltpu.async_copy(
            bc_v, out_hbm.at[pl.ds(blk, _L1), pl.ds(0, _HALF)], sem)
    pltpu.make_async_copy(
        out_hbm.at[pl.ds(base, _ROWS_W)],
        out_hbm.at[pl.ds(base, _ROWS_W)],
        sem,
    ).wait()


def kernel(seq_len_or_axial_dims, W0, b0, W1, b1, div0, mult0, div1, mult1):
    w0 = W0.reshape(1, _HALF)
    b0r = b0.reshape(1, _HALF)
    w1 = W1.reshape(1, _HALF)
    b1r = b1.reshape(1, _HALF)
    scal = jnp.stack([div0, mult0, div1, mult1]).reshape(1, 4)

    emb0, emb1 = pl.pallas_call(
        _tables_kernel,
        out_shape=[
            jax.ShapeDtypeStruct((_L0, _HALF), jnp.float32),
            jax.ShapeDtypeStruct((_L1, _HALF), jnp.float32),
        ],
    )(w0, b0r, w1, b1r, scal)

    expand = pl.kernel(
        _sc_expand,
        out_type=jax.ShapeDtypeStruct((_L0 * _L1, _DIM), jnp.float32),
        mesh=plsc.VectorSubcoreMesh(core_axis_name="c", subcore_axis_name="s"),
        scratch_types=[
            pltpu.VMEM((_C0W, _HALF), jnp.float32),
            pltpu.VMEM((_L1, _HALF), jnp.float32),
            pltpu.VMEM((_L1, _HALF), jnp.float32),
            pltpu.VMEM_SHARED((_L1, _HALF), jnp.float32),
            pltpu.SemaphoreType.DMA,
        ],
    )
    return expand(emb0, emb1)
